# baseline (speedup 1.0000x reference)
import functools

import jax
import jax.numpy as jnp
from jax import lax
from jax.experimental import pallas as pl
from jax.experimental.pallas import tpu as pltpu
from jax.experimental.pallas import tpu_sc as plsc

_NUM_CORES = 2
_NUM_SUBCORES = 16
_NUM_WORKERS = _NUM_CORES * _NUM_SUBCORES

_LANES = 128
_CHUNK = 40


@functools.partial(jax.jit, static_argnames=("rows_per_worker", "splits"))
def _sc_gather(table128, subrow_ids, rows_per_worker, splits):
    num_subrows = subrow_ids.shape[0]
    sub_per_worker = rows_per_worker * splits
    sub_chunk = _CHUNK * splits
    num_chunks = rows_per_worker // _CHUNK
    num_groups = num_chunks // 2
    mesh = plsc.VectorSubcoreMesh(
        core_axis_name="c",
        subcore_axis_name="s",
        num_cores=_NUM_CORES,
        num_subcores=_NUM_SUBCORES,
    )

    @functools.partial(
        pl.kernel,
        out_type=jax.ShapeDtypeStruct((num_subrows, _LANES), jnp.float32),
        mesh=mesh,
        scratch_types=[
            pltpu.VMEM((sub_per_worker,), jnp.int32),
            pltpu.VMEM((2, sub_chunk, _LANES), jnp.float32),
            pltpu.SemaphoreType.DMA,
            pltpu.SemaphoreType.DMA,
        ],
    )
    def k(table_hbm, idx_hbm, out_hbm, idx_v, buf_v, gsem0, gsem1):
        gsems = (gsem0, gsem1)
        wid = lax.axis_index("s") * _NUM_CORES + lax.axis_index("c")
        base = wid * sub_per_worker
        pltpu.sync_copy(idx_hbm.at[pl.ds(base, sub_per_worker)], idx_v)

        def start_gather(c, b):
            idx_chunk = idx_v.at[pl.ds(c * sub_chunk, sub_chunk)]
            pltpu.async_copy(table_hbm.at[idx_chunk], buf_v.at[b], gsems[b])

        def wait_gather(c, b):
            idx_chunk = idx_v.at[pl.ds(c * sub_chunk, sub_chunk)]
            pltpu.make_async_copy(
                table_hbm.at[idx_chunk], buf_v.at[b], gsems[b]).wait()

        def scatter(c, b):
            off = pl.multiple_of(base + c * sub_chunk, 8)
            pltpu.sync_copy(buf_v.at[b], out_hbm.at[pl.ds(off, sub_chunk)])

        start_gather(0, 0)

        def body(g, _):
            c = 2 * g
            start_gather(c + 1, 1)
            wait_gather(c, 0)
            scatter(c, 0)
            start_gather(c + 2, 0)
            wait_gather(c + 1, 1)
            scatter(c + 1, 1)
            return _

        lax.fori_loop(0, num_groups - 1, body, None)

        c = 2 * (num_groups - 1)
        start_gather(c + 1, 1)
        wait_gather(c, 0)
        scatter(c, 0)
        wait_gather(c + 1, 1)
        scatter(c + 1, 1)

    return k(table128, subrow_ids)


def kernel(relation_ids, embedding):
    b, n = relation_ids.shape
    d = embedding.shape[1]
    splits = d // _LANES
    num_rows = b * n
    assert num_rows % (_NUM_WORKERS * _CHUNK) == 0 and d % _LANES == 0
    table128 = embedding.astype(jnp.float32).reshape(-1, _LANES)
    flat_ids = relation_ids.reshape(-1).astype(jnp.int32)
    sub_ids = (flat_ids[:, None] * splits
               + jnp.arange(splits, dtype=jnp.int32)[None, :]).reshape(-1)
    out = _sc_gather(table128, sub_ids, num_rows // _NUM_WORKERS, splits)
    return out.reshape(b, n, d)
